# baseline (speedup 1.0000x reference)
import functools

import jax
import jax.numpy as jnp
from jax import lax
from jax.experimental import pallas as pl
from jax.experimental.pallas import tpu as pltpu
from jax.experimental.pallas import tpu_sc as plsc

N_CLASSES = 1000
BATCH = 16384
_L = 16

_NC = 2
_NS = 16
_NW = _NC * _NS
_ROWS_PER_W = BATCH // _NW
_ZROWS = 32
_ZWORDS = _ZROWS * N_CLASSES
_NZB = 4
_NZD = _ROWS_PER_W // _ZROWS
_G = _ROWS_PER_W // _L
_ICHUNK = 128
_NI = _ROWS_PER_W // _ICHUNK


def _onehot_body(p_hbm, out_hbm, p_v, z0, offs, ones_v, *sems):
    zsems = sems[:_NZB]
    isems = sems[_NZB:]
    wid = lax.axis_index("s") * _NC + lax.axis_index("c")
    base = wid * _ROWS_PER_W
    pltpu.sync_copy(p_hbm.at[pl.ds(base, _ROWS_PER_W)], p_v)

    zeros16 = jnp.zeros((_L,), jnp.float32)
    ones16 = jnp.ones((_L,), jnp.float32)
    rows16 = lax.iota(jnp.int32, _L)

    def zbody(i, carry):
        for u in range(16):
            z0[pl.ds(i * (16 * _L) + u * _L, _L)] = zeros16
        return carry
    lax.fori_loop(0, _ZWORDS // (16 * _L), zbody, 0)

    wbase = base * N_CLASSES
    zcopies = [
        pltpu.async_copy(
            z0,
            out_hbm.at[pl.ds(wbase + k * _ZWORDS, _ZWORDS)],
            zsems[k % _NZB],
        )
        for k in range(_NZD)
    ]

    for u in range(_ICHUNK // _L):
        ones_v[pl.ds(u * _L, _L)] = ones16
    for u in range(_G):
        cols = p_v[pl.ds(u * _L, _L)]
        off16 = (rows16 + (base + u * _L)) * N_CLASSES + cols
        offs[u // (_ICHUNK // _L), pl.ds((u % (_ICHUNK // _L)) * _L, _L)] = off16

    for c in zcopies:
        c.wait()

    icopies = [
        pltpu.async_copy(ones_v, out_hbm.at[offs.at[j]], isems[j])
        for j in range(_NI)
    ]
    for c in icopies:
        c.wait()


def kernel(p, class_matrix):
    del class_matrix
    mesh = plsc.VectorSubcoreMesh(core_axis_name="c", subcore_axis_name="s")
    run = functools.partial(
        pl.kernel,
        mesh=mesh,
        out_type=jax.ShapeDtypeStruct((BATCH * N_CLASSES,), jnp.float32),
        scratch_types=[
            pltpu.VMEM((_ROWS_PER_W,), jnp.int32),
            pltpu.VMEM((_ZWORDS,), jnp.float32),
            pltpu.VMEM((_NI, _ICHUNK), jnp.int32),
            pltpu.VMEM((_ICHUNK,), jnp.float32),
        ] + [pltpu.SemaphoreType.DMA] * (_NZB + _NI),
        compiler_params=pltpu.CompilerParams(needs_layout_passes=False),
    )(_onehot_body)
    return run(p.astype(jnp.int32)).reshape(BATCH, N_CLASSES)


# speedup vs baseline: 1.9431x; 1.9431x over previous
import functools

import jax
import jax.numpy as jnp
from jax import lax
from jax.experimental import pallas as pl
from jax.experimental.pallas import tpu as pltpu
from jax.experimental.pallas import tpu_sc as plsc

N_CLASSES = 1000
_W = 1024
BATCH = 16384
_L = 16

_NC = 2
_NS = 16
_NW = _NC * _NS
_ROWS_PER_W = BATCH // _NW
_C = 16
_NCHUNK = _ROWS_PER_W // _C
_NBUF = 4


def _onehot_body(p_hbm, out_hbm, p_v, b0, b1, b2, b3, s0, s1, s2, s3):
    bufs = (b0, b1, b2, b3)
    sems = (s0, s1, s2, s3)
    wid = lax.axis_index("s") * _NC + lax.axis_index("c")
    base = wid * _ROWS_PER_W
    pltpu.sync_copy(p_hbm.at[pl.ds(base, _ROWS_PER_W)], p_v)

    zeros16 = jnp.zeros((_L,), jnp.float32)
    ones16 = jnp.ones((_L,), jnp.float32)
    rows16 = lax.iota(jnp.int32, _L)

    def body(i, carry):
        for u in range(_W // _L):
            b0[i, pl.ds(u * _L, _L)] = zeros16
        return carry
    lax.fori_loop(0, _C, body, 0)
    for b in range(1, _NBUF):
        def bodyb(i, carry, _b=b):
            for u in range(_W // _L):
                bufs[_b][i, pl.ds(u * _L, _L)] = zeros16
            return carry
        lax.fori_loop(0, _C, bodyb, 0)

    def fire(b, off):
        for g in range(_C // _L):
            cols = p_v[pl.ds(off + g * _L, _L)]
            plsc.store_scatter(bufs[b], [rows16 + (g * _L), cols], ones16)
        pltpu.async_copy(bufs[b], out_hbm.at[pl.ds(base + off, _C)], sems[b])

    def drain(b):
        pltpu.make_async_copy(bufs[b], out_hbm.at[pl.ds(0, _C)], sems[b]).wait()

    for b in range(_NBUF):
        fire(b, b * _C)

    def group(gg, carry):
        off0 = gg * _NBUF * _C
        for b in range(_NBUF):
            off = off0 + b * _C
            drain(b)
            for g in range(_C // _L):
                old_cols = p_v[pl.ds(off - _NBUF * _C + g * _L, _L)]
                plsc.store_scatter(bufs[b], [rows16 + (g * _L), old_cols], zeros16)
            fire(b, off)
        return carry

    lax.fori_loop(1, _NCHUNK // _NBUF, group, 0)
    for b in range(_NBUF):
        drain(b)


def kernel(p, class_matrix):
    del class_matrix
    mesh = plsc.VectorSubcoreMesh(core_axis_name="c", subcore_axis_name="s")
    run = functools.partial(
        pl.kernel,
        mesh=mesh,
        out_type=jax.ShapeDtypeStruct((BATCH, _W), jnp.float32),
        scratch_types=[
            pltpu.VMEM((_ROWS_PER_W,), jnp.int32),
        ] + [pltpu.VMEM((_C, _W), jnp.float32)] * _NBUF
          + [pltpu.SemaphoreType.DMA] * _NBUF,
        compiler_params=pltpu.CompilerParams(needs_layout_passes=False),
    )(_onehot_body)
    return run(p.astype(jnp.int32))[:, :N_CLASSES]


# speedup vs baseline: 1.9570x; 1.0071x over previous
import functools

import jax
import jax.numpy as jnp
from jax import lax
from jax.experimental import pallas as pl
from jax.experimental.pallas import tpu as pltpu
from jax.experimental.pallas import tpu_sc as plsc

N_CLASSES = 1000
_W = 1024
BATCH = 16384
_L = 16

_NC = 2
_NS = 16
_NW = _NC * _NS
_ROWS_PER_W = BATCH // _NW
_C = 32
_NCHUNK = _ROWS_PER_W // _C
_NBUF = 2


def _onehot_body(p_hbm, out_hbm, p_v, b0, b1, s0, s1):
    bufs = (b0, b1)
    sems = (s0, s1)
    wid = lax.axis_index("s") * _NC + lax.axis_index("c")
    base = wid * _ROWS_PER_W
    pltpu.sync_copy(p_hbm.at[pl.ds(base, _ROWS_PER_W)], p_v)

    zeros16 = jnp.zeros((_L,), jnp.float32)
    ones16 = jnp.ones((_L,), jnp.float32)
    rows16 = lax.iota(jnp.int32, _L)

    def body(i, carry):
        for u in range(_W // _L):
            b0[i, pl.ds(u * _L, _L)] = zeros16
        return carry
    lax.fori_loop(0, _C, body, 0)
    for b in range(1, _NBUF):
        def bodyb(i, carry, _b=b):
            for u in range(_W // _L):
                bufs[_b][i, pl.ds(u * _L, _L)] = zeros16
            return carry
        lax.fori_loop(0, _C, bodyb, 0)

    def fire(b, off):
        for g in range(_C // _L):
            cols = p_v[pl.ds(off + g * _L, _L)]
            plsc.store_scatter(bufs[b], [rows16 + (g * _L), cols], ones16)
        pltpu.async_copy(bufs[b], out_hbm.at[pl.ds(base + off, _C)], sems[b])

    def drain(b):
        pltpu.make_async_copy(bufs[b], out_hbm.at[pl.ds(0, _C)], sems[b]).wait()

    for b in range(_NBUF):
        fire(b, b * _C)

    def group(gg, carry):
        off0 = gg * _NBUF * _C
        for b in range(_NBUF):
            off = off0 + b * _C
            drain(b)
            for g in range(_C // _L):
                old_cols = p_v[pl.ds(off - _NBUF * _C + g * _L, _L)]
                plsc.store_scatter(bufs[b], [rows16 + (g * _L), old_cols], zeros16)
            fire(b, off)
        return carry

    lax.fori_loop(1, _NCHUNK // _NBUF, group, 0)
    for b in range(_NBUF):
        drain(b)


def kernel(p, class_matrix):
    del class_matrix
    mesh = plsc.VectorSubcoreMesh(core_axis_name="c", subcore_axis_name="s")
    run = functools.partial(
        pl.kernel,
        mesh=mesh,
        out_type=jax.ShapeDtypeStruct((BATCH, _W), jnp.float32),
        scratch_types=[
            pltpu.VMEM((_ROWS_PER_W,), jnp.int32),
        ] + [pltpu.VMEM((_C, _W), jnp.float32)] * _NBUF
          + [pltpu.SemaphoreType.DMA] * _NBUF,
        compiler_params=pltpu.CompilerParams(needs_layout_passes=False),
    )(_onehot_body)
    return run(p.astype(jnp.int32))[:, :N_CLASSES]
